# baseline (speedup 1.0000x reference)
import functools
import math

import jax
import jax.numpy as jnp
from jax import lax
from jax.experimental import pallas as pl
from jax.experimental.pallas import tpu as pltpu
from jax.experimental.pallas import tpu_sc as plsc

_N = 524288
_N_IMAGES = 4096
_N_REFL = 65536
_H = 32
_S = 32
_BLK = 256
_NBLK = _N // _BLK
_TB = 2048
_ACC_ROWS = _N_IMAGES + _BLK
_NCOL = 16
_LOG2PI = math.log(2.0 * math.pi)

_SC_WORKERS = 32
_SC_CHUNK = 32768


def _softplus(x):
    return jnp.maximum(x, 0.0) + jnp.log1p(jnp.exp(-jnp.abs(x)))


def _table_body(eps_ref, ql_ref, qr_ref, tab_ref, kl_ref):
    e = eps_ref[...]
    ql = ql_ref[...]
    qs = _softplus(qr_ref[...])
    e1 = jnp.sum(e, axis=0, keepdims=True)
    e2 = jnp.sum(e * e, axis=0, keepdims=True)
    s1 = _S * ql + qs * e1
    s2 = _S * ql * ql + 2.0 * ql * qs * e1 + qs * qs * e2
    tab_ref[...] = jnp.concatenate([ql, qs, s1, s2], axis=0)
    klt = -jnp.log(qs) + 0.5 * (qs * qs + ql * ql) - 0.5

    @pl.when(pl.program_id(0) == 0)
    def _():
        kl_ref[...] = jnp.zeros_like(kl_ref)

    kl_ref[...] += jnp.sum(klt).reshape(1, 1)


def _build_table(eps, q_loc, q_raw_scale):
    grid = _N_REFL // _TB
    tab4, klsum = pl.pallas_call(
        _table_body,
        grid=(grid,),
        in_specs=[
            pl.BlockSpec((_S, _TB), lambda i: (0, i)),
            pl.BlockSpec((1, _TB), lambda i: (0, i)),
            pl.BlockSpec((1, _TB), lambda i: (0, i)),
        ],
        out_specs=[
            pl.BlockSpec((4, _TB), lambda i: (0, i)),
            pl.BlockSpec((1, 1), lambda i: (0, 0)),
        ],
        out_shape=[
            jax.ShapeDtypeStruct((4, _N_REFL), jnp.float32),
            jax.ShapeDtypeStruct((1, 1), jnp.float32),
        ],
    )(eps, q_loc.reshape(1, _N_REFL), q_raw_scale.reshape(1, _N_REFL))
    return tab4, klsum


def _sc_gather(table_lin, idx4):
    m4 = idx4.shape[0]
    per_w = m4 // _SC_WORKERS
    n_chunks = per_w // _SC_CHUNK
    mesh = plsc.VectorSubcoreMesh(core_axis_name="c", subcore_axis_name="s")

    @functools.partial(
        pl.kernel,
        mesh=mesh,
        out_type=jax.ShapeDtypeStruct((m4,), jnp.float32),
        scratch_types=[
            pltpu.VMEM((_SC_CHUNK,), jnp.int32),
            pltpu.VMEM((_SC_CHUNK,), jnp.float32),
            pltpu.VMEM_SHARED((_N_REFL * 4,), jnp.float32),
            pltpu.SemaphoreType.DMA,
        ],
    )
    def k(table_hbm, idx_hbm, out_hbm, idx_v, rows_v, tab_sh, sem):
        c = lax.axis_index("c")
        s = lax.axis_index("s")
        wid = s * 2 + c
        base = wid * per_w

        @pl.when(s == 0)
        def _():
            pltpu.sync_copy(table_hbm, tab_sh)

        plsc.subcore_barrier()

        def body(i, carry):
            off = base + i * _SC_CHUNK
            pltpu.sync_copy(idx_hbm.at[pl.ds(off, _SC_CHUNK)], idx_v)
            pltpu.async_copy(tab_sh.at[idx_v], rows_v, sem).wait()
            pltpu.sync_copy(rows_v, out_hbm.at[pl.ds(off, _SC_CHUNK)])
            return carry

        lax.fori_loop(0, n_chunks, body, 0)

    return k(table_lin, idx4)


def _main_body(gp_ref, gm_ref, obs_ref, seg_ref, base_ref, img_ref,
               w1_ref, b1_ref, w2_ref, b2_ref, acc_ref):
    pid = pl.program_id(0)
    base = base_ref[pid]

    segs = seg_ref[0]
    local = segs - base
    iota0 = lax.broadcasted_iota(jnp.int32, (_BLK, _BLK), 0)
    q = (local == iota0).astype(jnp.float32)

    img_slice = img_ref[pl.ds(base, _BLK), :]
    img_obs = lax.dot_general(q, img_slice, (((0,), (0,)), ((), ())),
                              preferred_element_type=jnp.float32)

    ob = obs_ref[...]
    xi = ob[:, 0:1]
    sig = ob[:, 1:2]
    gp = gp_ref[0]
    gm = gm_ref[0]
    w1 = w1_ref[...]
    b1 = b1_ref[...]
    w2 = w2_ref[...]
    b2 = b2_ref[...]

    def mlp_scale(g):
        feat = jnp.concatenate([g[:, 0:2], ob, img_obs], axis=1)
        h = jnp.tanh(jnp.dot(feat, w1, preferred_element_type=jnp.float32) + b1)
        r = jnp.dot(h, w2, preferred_element_type=jnp.float32) + b2
        return _softplus(r)

    scp = mlp_scale(gp)
    scm = mlp_scale(gm)

    w = 1.0 / (sig * sig)
    lsig = jnp.log(sig)

    def ll_row(sc, g):
        s1 = g[:, 2:3]
        s2 = g[:, 3:4]
        quad = sc * sc * s2 - 2.0 * xi * sc * s1 + _S * xi * xi
        return -0.5 * w * quad - _S * lsig - (_S / 2.0) * _LOG2PI

    llp = ll_row(scp, gp)
    llm = ll_row(scm, gm)
    yp = scp * gp[:, 2:3] * (1.0 / _S)
    ym = scm * gm[:, 2:3] * (1.0 / _S)

    zero = jnp.zeros((_BLK, _NCOL - 11), jnp.float32)
    vals = jnp.concatenate(
        [llp, llm, w * yp, w * ym, w * yp * yp, w * ym * ym,
         w * xi * yp, w * xi * ym, w, w * xi, w * xi * xi, zero], axis=1)

    part = lax.dot_general(q, vals, (((1,), (0,)), ((), ())),
                           preferred_element_type=jnp.float32,
                           precision=lax.Precision.HIGHEST)

    @pl.when(pid == 0)
    def _():
        acc_ref[...] = jnp.zeros_like(acc_ref)

    acc_ref[pl.ds(base, _BLK), :] += part


def _main_pass(g2, obs, seg3, seg_base, img_pad, w1, b1, w2, b2):
    acc = pl.pallas_call(
        _main_body,
        grid=(_NBLK,),
        in_specs=[
            pl.BlockSpec((1, _BLK, 4), lambda i: (0, i, 0)),
            pl.BlockSpec((1, _BLK, 4), lambda i: (1, i, 0)),
            pl.BlockSpec((_BLK, 8), lambda i: (i, 0)),
            pl.BlockSpec((1, 1, _BLK), lambda i: (i, 0, 0)),
            pl.BlockSpec(memory_space=pltpu.SMEM),
            pl.BlockSpec((_ACC_ROWS, 8), lambda i: (0, 0)),
            pl.BlockSpec((18, _H), lambda i: (0, 0)),
            pl.BlockSpec((1, _H), lambda i: (0, 0)),
            pl.BlockSpec((_H, 1), lambda i: (0, 0)),
            pl.BlockSpec((1, 1), lambda i: (0, 0)),
        ],
        out_specs=pl.BlockSpec((_ACC_ROWS, _NCOL), lambda i: (0, 0)),
        out_shape=jax.ShapeDtypeStruct((_ACC_ROWS, _NCOL), jnp.float32),
    )(g2, g2, obs, seg3, seg_base, img_pad, w1, b1, w2, b2)
    return acc


def kernel(hkl, I, SigI, image_id, metadata, wavelength, dHKL, mc_samples,
           q_loc, q_raw_scale, img_emb, W1, b1, W2, b2):
    h = hkl[:, 0].astype(jnp.int32)
    k = hkl[:, 1].astype(jnp.int32)
    l = hkl[:, 2].astype(jnp.int32)
    sh = h * 1000003 + k * 10007 + l * 101
    rp = jnp.mod(sh, _N_REFL).astype(jnp.int32)
    rm = jnp.mod(-sh, _N_REFL).astype(jnp.int32)
    idx_all = jnp.concatenate([rp, rm], axis=0)

    im = image_id.astype(jnp.int32)
    step = (im[1:] != im[:-1]).astype(jnp.int32)
    seg = jnp.concatenate([jnp.zeros((1,), jnp.int32), jnp.cumsum(step)])
    seg = seg.astype(jnp.int32)
    seg_base = seg[:: _BLK]
    seg3 = seg.reshape(_NBLK, 1, _BLK)

    obs = jnp.concatenate(
        [I, SigI, metadata, wavelength, 1.0 / jnp.square(dHKL)], axis=1)

    img_pad = jnp.zeros((_ACC_ROWS, 8), jnp.float32).at[:_N_IMAGES].set(img_emb)

    eps = jax.random.normal(jax.random.key(42), (_S, _N_REFL), dtype=jnp.float32)
    tab4, klsum = _build_table(eps, q_loc, q_raw_scale)
    table = tab4.T
    kl_div = klsum[0, 0] / _N_REFL

    idx4 = (idx_all[:, None] * 4 + jnp.arange(4, dtype=jnp.int32)[None, :]).reshape(-1)
    g = table.reshape(-1)[idx4]
    g2 = g.reshape(2, _N, 4)

    acc = _main_pass(g2, obs, seg3, seg_base, img_pad,
                     W1, b1.reshape(1, _H), W2, b2.reshape(1, 1))
    a = acc[:_N_IMAGES]

    llp = a[:, 0] / _S
    llm = a[:, 1] / _S
    op_idx = (llm > llp).astype(jnp.int32)
    ll_max = jnp.maximum(llp, llm)
    elbo = -jnp.mean(ll_max) + kl_div

    sel = op_idx.astype(bool)
    swy = jnp.sum(jnp.where(sel, a[:, 3], a[:, 2]))
    swy2 = jnp.sum(jnp.where(sel, a[:, 5], a[:, 4]))
    swxy = jnp.sum(jnp.where(sel, a[:, 7], a[:, 6]))
    sw = jnp.sum(a[:, 8])
    swx = jnp.sum(a[:, 9])
    swx2 = jnp.sum(a[:, 10])
    zi = 1.0 / sw
    mx = zi * swx
    my = zi * swy
    cxy = zi * swxy - mx * my
    cx = zi * swx2 - mx * mx
    cy = zi * swy2 - my * my
    cc = cxy / jnp.sqrt(cx * cy)

    return elbo, cc, op_idx


# speedup vs baseline: 10.2050x; 10.2050x over previous
import functools
import math

import jax
import jax.numpy as jnp
from jax import lax
from jax.experimental import pallas as pl
from jax.experimental.pallas import tpu as pltpu
from jax.experimental.pallas import tpu_sc as plsc

_N = 524288
_N_IMAGES = 4096
_N_REFL = 65536
_H = 32
_S = 32
_BLK = 256
_NBLK = _N // _BLK
_TB = 2048
_ACC_ROWS = _N_IMAGES + _BLK
_NCOL = 16
_LOG2PI = math.log(2.0 * math.pi)

_SC_WORKERS = 32
_SC_CHUNK = 32768


def _softplus(x):
    return jnp.maximum(x, 0.0) + jnp.log1p(jnp.exp(-jnp.abs(x)))


def _table_body(eps_ref, ql_ref, qr_ref, tab_ref, kl_ref):
    e = eps_ref[...]
    ql = ql_ref[...]
    qs = _softplus(qr_ref[...])
    e1 = jnp.sum(e, axis=0, keepdims=True)
    e2 = jnp.sum(e * e, axis=0, keepdims=True)
    s1 = _S * ql + qs * e1
    s2 = _S * ql * ql + 2.0 * ql * qs * e1 + qs * qs * e2
    tab_ref[...] = jnp.concatenate([ql, qs, s1, s2], axis=0)
    klt = -jnp.log(qs) + 0.5 * (qs * qs + ql * ql) - 0.5

    @pl.when(pl.program_id(0) == 0)
    def _():
        kl_ref[...] = jnp.zeros_like(kl_ref)

    kl_ref[...] += jnp.sum(klt).reshape(1, 1)


def _build_table(eps, q_loc, q_raw_scale):
    grid = _N_REFL // _TB
    tab4, klsum = pl.pallas_call(
        _table_body,
        grid=(grid,),
        in_specs=[
            pl.BlockSpec((_S, _TB), lambda i: (0, i)),
            pl.BlockSpec((1, _TB), lambda i: (0, i)),
            pl.BlockSpec((1, _TB), lambda i: (0, i)),
        ],
        out_specs=[
            pl.BlockSpec((4, _TB), lambda i: (0, i)),
            pl.BlockSpec((1, 1), lambda i: (0, 0)),
        ],
        out_shape=[
            jax.ShapeDtypeStruct((4, _N_REFL), jnp.float32),
            jax.ShapeDtypeStruct((1, 1), jnp.float32),
        ],
    )(eps, q_loc.reshape(1, _N_REFL), q_raw_scale.reshape(1, _N_REFL))
    return tab4, klsum


def _sc_gather(table_lin, idx4):
    m4 = idx4.shape[0]
    per_w = m4 // _SC_WORKERS
    n_chunks = per_w // _SC_CHUNK
    mesh = plsc.VectorSubcoreMesh(core_axis_name="c", subcore_axis_name="s")

    @functools.partial(
        pl.kernel,
        mesh=mesh,
        out_type=jax.ShapeDtypeStruct((m4,), jnp.float32),
        scratch_types=[
            pltpu.VMEM((_SC_CHUNK,), jnp.int32),
            pltpu.VMEM((_SC_CHUNK,), jnp.float32),
            pltpu.VMEM_SHARED((_N_REFL * 4,), jnp.float32),
            pltpu.SemaphoreType.DMA,
        ],
    )
    def k(table_hbm, idx_hbm, out_hbm, idx_v, rows_v, tab_sh, sem):
        c = lax.axis_index("c")
        s = lax.axis_index("s")
        wid = s * 2 + c
        base = wid * per_w

        @pl.when(s == 0)
        def _():
            pltpu.sync_copy(table_hbm, tab_sh)

        plsc.subcore_barrier()

        def body(i, carry):
            off = base + i * _SC_CHUNK
            pltpu.sync_copy(idx_hbm.at[pl.ds(off, _SC_CHUNK)], idx_v)
            pltpu.async_copy(tab_sh.at[idx_v], rows_v, sem).wait()
            pltpu.sync_copy(rows_v, out_hbm.at[pl.ds(off, _SC_CHUNK)])
            return carry

        lax.fori_loop(0, n_chunks, body, 0)

    return k(table_lin, idx4)


def _main_body(gp_ref, gm_ref, obs_ref, seg_ref, base_ref, img_ref,
               w1_ref, b1_ref, w2_ref, b2_ref, acc_ref):
    pid = pl.program_id(0)
    base = base_ref[pid]

    segs = seg_ref[0]
    local = segs - base
    iota0 = lax.broadcasted_iota(jnp.int32, (_BLK, _BLK), 0)
    q = (local == iota0).astype(jnp.float32)

    img_slice = img_ref[pl.ds(base, _BLK), :]
    img_obs = lax.dot_general(q, img_slice, (((0,), (0,)), ((), ())),
                              preferred_element_type=jnp.float32)

    ob = obs_ref[...]
    xi = ob[:, 0:1]
    sig = ob[:, 1:2]
    gp = gp_ref[0]
    gm = gm_ref[0]
    w1 = w1_ref[...]
    b1 = b1_ref[...]
    w2 = w2_ref[...]
    b2 = b2_ref[...]

    def mlp_scale(g):
        feat = jnp.concatenate([g[:, 0:2], ob, img_obs], axis=1)
        h = jnp.tanh(jnp.dot(feat, w1, preferred_element_type=jnp.float32) + b1)
        r = jnp.dot(h, w2, preferred_element_type=jnp.float32) + b2
        return _softplus(r)

    scp = mlp_scale(gp)
    scm = mlp_scale(gm)

    w = 1.0 / (sig * sig)
    lsig = jnp.log(sig)

    def ll_row(sc, g):
        s1 = g[:, 2:3]
        s2 = g[:, 3:4]
        quad = sc * sc * s2 - 2.0 * xi * sc * s1 + _S * xi * xi
        return -0.5 * w * quad - _S * lsig - (_S / 2.0) * _LOG2PI

    llp = ll_row(scp, gp)
    llm = ll_row(scm, gm)
    yp = scp * gp[:, 2:3] * (1.0 / _S)
    ym = scm * gm[:, 2:3] * (1.0 / _S)

    zero = jnp.zeros((_BLK, _NCOL - 11), jnp.float32)
    vals = jnp.concatenate(
        [llp, llm, w * yp, w * ym, w * yp * yp, w * ym * ym,
         w * xi * yp, w * xi * ym, w, w * xi, w * xi * xi, zero], axis=1)

    vhi = vals.astype(jnp.bfloat16).astype(jnp.float32)
    vlo = vals - vhi
    dn = (((1,), (0,)), ((), ()))
    part = (lax.dot_general(q, vhi, dn, preferred_element_type=jnp.float32)
            + lax.dot_general(q, vlo, dn, preferred_element_type=jnp.float32))

    @pl.when(pid == 0)
    def _():
        acc_ref[...] = jnp.zeros_like(acc_ref)

    acc_ref[pl.ds(base, _BLK), :] += part


def _main_pass(g2, obs, seg3, seg_base, img_pad, w1, b1, w2, b2):
    acc = pl.pallas_call(
        _main_body,
        grid=(_NBLK,),
        in_specs=[
            pl.BlockSpec((1, _BLK, 4), lambda i: (0, i, 0)),
            pl.BlockSpec((1, _BLK, 4), lambda i: (1, i, 0)),
            pl.BlockSpec((_BLK, 8), lambda i: (i, 0)),
            pl.BlockSpec((1, 1, _BLK), lambda i: (i, 0, 0)),
            pl.BlockSpec(memory_space=pltpu.SMEM),
            pl.BlockSpec((_ACC_ROWS, 8), lambda i: (0, 0)),
            pl.BlockSpec((18, _H), lambda i: (0, 0)),
            pl.BlockSpec((1, _H), lambda i: (0, 0)),
            pl.BlockSpec((_H, 1), lambda i: (0, 0)),
            pl.BlockSpec((1, 1), lambda i: (0, 0)),
        ],
        out_specs=pl.BlockSpec((_ACC_ROWS, _NCOL), lambda i: (0, 0)),
        out_shape=jax.ShapeDtypeStruct((_ACC_ROWS, _NCOL), jnp.float32),
    )(g2, g2, obs, seg3, seg_base, img_pad, w1, b1, w2, b2)
    return acc


def kernel(hkl, I, SigI, image_id, metadata, wavelength, dHKL, mc_samples,
           q_loc, q_raw_scale, img_emb, W1, b1, W2, b2):
    h = hkl[:, 0].astype(jnp.int32)
    k = hkl[:, 1].astype(jnp.int32)
    l = hkl[:, 2].astype(jnp.int32)
    sh = h * 1000003 + k * 10007 + l * 101
    rp = jnp.mod(sh, _N_REFL).astype(jnp.int32)
    rm = jnp.mod(-sh, _N_REFL).astype(jnp.int32)
    idx_all = jnp.concatenate([rp, rm], axis=0)

    im = image_id.astype(jnp.int32)
    step = (im[1:] != im[:-1]).astype(jnp.int32)
    seg = jnp.concatenate([jnp.zeros((1,), jnp.int32), jnp.cumsum(step)])
    seg = seg.astype(jnp.int32)
    seg_base = seg[:: _BLK]
    seg3 = seg.reshape(_NBLK, 1, _BLK)

    obs = jnp.concatenate(
        [I, SigI, metadata, wavelength, 1.0 / jnp.square(dHKL)], axis=1)

    img_pad = jnp.zeros((_ACC_ROWS, 8), jnp.float32).at[:_N_IMAGES].set(img_emb)

    eps = jax.random.normal(jax.random.key(42), (_S, _N_REFL), dtype=jnp.float32)
    tab4, klsum = _build_table(eps, q_loc, q_raw_scale)
    table = tab4.T
    kl_div = klsum[0, 0] / _N_REFL

    idx4 = (idx_all[:, None] * 4 + jnp.arange(4, dtype=jnp.int32)[None, :]).reshape(-1)
    g = _sc_gather(table.reshape(-1), idx4)
    g2 = g.reshape(2, _N, 4)

    acc = _main_pass(g2, obs, seg3, seg_base, img_pad,
                     W1, b1.reshape(1, _H), W2, b2.reshape(1, 1))
    a = acc[:_N_IMAGES]

    llp = a[:, 0] / _S
    llm = a[:, 1] / _S
    op_idx = (llm > llp).astype(jnp.int32)
    ll_max = jnp.maximum(llp, llm)
    elbo = -jnp.mean(ll_max) + kl_div

    sel = op_idx.astype(bool)
    swy = jnp.sum(jnp.where(sel, a[:, 3], a[:, 2]))
    swy2 = jnp.sum(jnp.where(sel, a[:, 5], a[:, 4]))
    swxy = jnp.sum(jnp.where(sel, a[:, 7], a[:, 6]))
    sw = jnp.sum(a[:, 8])
    swx = jnp.sum(a[:, 9])
    swx2 = jnp.sum(a[:, 10])
    zi = 1.0 / sw
    mx = zi * swx
    my = zi * swy
    cxy = zi * swxy - mx * my
    cx = zi * swx2 - mx * mx
    cy = zi * swy2 - my * my
    cc = cxy / jnp.sqrt(cx * cy)

    return elbo, cc, op_idx
